# baseline (speedup 1.0000x reference)
import functools

import jax
import jax.numpy as jnp
from jax import lax
from jax.experimental import pallas as pl
from jax.experimental.pallas import tpu as pltpu
from jax.experimental.pallas import tpu_sc as plsc

N_ROWS = 16384
NC = 1000
VAL = 1001.0
NUM_WORKERS = 32
COLS_PER_WORKER = N_ROWS // NUM_WORKERS
CHUNK_J = 40
N_CHUNKS = NC // CHUNK_J


def _fill_ones(buf):
    ones16 = jnp.full((16,), 1.0, jnp.float32)

    def row_body(r, _):
        def col_body(c, _):
            buf[r, pl.ds(c * 16, 16)] = ones16
            return 0

        lax.fori_loop(0, COLS_PER_WORKER // 16, col_body, 0)
        return 0

    lax.fori_loop(0, CHUNK_J, row_body, 0)


def _poke(buf, lab, chunk, value):
    iota = lax.iota(jnp.int32, 16)
    vals = jnp.full((16,), value, jnp.float32)
    j0 = chunk * CHUNK_J

    def body(v, _):
        labv = lab[pl.ds(v * 16, 16)]
        mask = (labv >= j0) & (labv < j0 + CHUNK_J)
        plsc.store_scatter(buf, [labv - j0, iota + v * 16], vals, mask=mask)
        return 0

    lax.fori_loop(0, COLS_PER_WORKER // 16, body, 0)


@functools.partial(
    pl.kernel,
    out_type=jax.ShapeDtypeStruct((NC, N_ROWS), jnp.float32),
    mesh=plsc.VectorSubcoreMesh(core_axis_name="c", subcore_axis_name="s"),
    compiler_params=pltpu.CompilerParams(needs_layout_passes=False),
    scratch_types=[
        pltpu.VMEM((CHUNK_J, COLS_PER_WORKER), jnp.float32),
        pltpu.VMEM((CHUNK_J, COLS_PER_WORKER), jnp.float32),
        pltpu.VMEM((COLS_PER_WORKER,), jnp.int32),
        pltpu.SemaphoreType.DMA,
        pltpu.SemaphoreType.DMA,
    ],
)
def _sc_smooth_onehot_t(labels_hbm, out_hbm, buf0, buf1, lab, sem0, sem1):
    wid = lax.axis_index("s") * 2 + lax.axis_index("c")
    col0 = pl.multiple_of(wid * COLS_PER_WORKER, 128)

    pltpu.sync_copy(labels_hbm.at[pl.ds(col0, COLS_PER_WORKER)], lab)

    bufs = (buf0, buf1)
    sems = (sem0, sem1)
    copies = [None, None]

    for k in range(N_CHUNKS):
        b = k % 2
        if k < 2:
            _fill_ones(bufs[b])
        else:
            copies[b].wait()
            _poke(bufs[b], lab, k - 2, 1.0)
        _poke(bufs[b], lab, k, VAL)
        dst = out_hbm.at[pl.ds(pl.multiple_of(k * CHUNK_J, 8), CHUNK_J),
                         pl.ds(col0, COLS_PER_WORKER)]
        copies[b] = pltpu.async_copy(bufs[b], dst, sems[b])

    copies[0].wait()
    copies[1].wait()


def kernel(labels):
    return _sc_smooth_onehot_t(labels.astype(jnp.int32)).T


# speedup vs baseline: 2.5116x; 2.5116x over previous
import jax
import jax.numpy as jnp
from jax.experimental import pallas as pl

N_ROWS = 16384
NC = 1000
VAL = 1001.0
COLS_PER_BLOCK = 1024


def _smooth_onehot_t_block(lab_ref, out_ref):
    lab = lab_ref[...]
    jrow = jax.lax.broadcasted_iota(jnp.int32, (NC, lab.shape[1]), 0)
    out_ref[...] = jnp.where(lab == jrow, VAL, 1.0)


def kernel(labels):
    c = COLS_PER_BLOCK
    lab2d = labels.astype(jnp.int32).reshape(1, N_ROWS)
    out_t = pl.pallas_call(
        _smooth_onehot_t_block,
        grid=(N_ROWS // c,),
        in_specs=[pl.BlockSpec((1, c), lambda i: (0, i))],
        out_specs=pl.BlockSpec((NC, c), lambda i: (0, i)),
        out_shape=jax.ShapeDtypeStruct((NC, N_ROWS), jnp.float32),
    )(lab2d)
    return out_t.T
